# baseline (speedup 1.0000x reference)
import functools

import jax
import jax.numpy as jnp
from jax import lax
from jax.experimental import pallas as pl
from jax.experimental.pallas import tpu as pltpu
from jax.experimental.pallas import tpu_sc as plsc

_B, _D, _L, _K = 9216, 768, 256, 8192
_BLK = 256
_NBLK = _B // _BLK

_NW = 32
_BPW = _B // _NW
_CH = 96
_NCH = _BPW // _CH


def _encode_vq_body(x_ref, w1_ref, b1_ref, w2_ref, b2_ref, w3_ref, b3_ref,
                    cb_ref, ze_ref, idx_ref, csum_ref):
    i = pl.program_id(0)

    @pl.when(i == 0)
    def _():
        cb = cb_ref[...]
        csum_ref[...] = jnp.sum(cb * cb, axis=1)[None, :]

    x = x_ref[...]
    h = jnp.maximum(jnp.dot(x, w1_ref[...]) + b1_ref[...], 0.0)
    h = jnp.maximum(jnp.dot(h, w2_ref[...]) + b2_ref[...], 0.0)
    z = jnp.dot(h, w3_ref[...]) + b3_ref[...]
    ze_ref[...] = z

    ab = lax.dot_general(z, cb_ref[...], (((1,), (1,)), ((), ())))
    rowsum = jnp.sum(z * z, axis=1, keepdims=True)
    d2 = (rowsum - 2.0 * ab) + csum_ref[...]
    m = jnp.min(d2, axis=1, keepdims=True)
    col = lax.broadcasted_iota(jnp.int32, (_BLK, _K), 1)
    idx = jnp.min(jnp.where(d2 == m, col, _K), axis=1)
    idx_ref[...] = idx[None, None, :]


def _encode_vq(x, w1, b1, w2, b2, w3, b3, cb):
    return pl.pallas_call(
        _encode_vq_body,
        grid=(_NBLK,),
        in_specs=[
            pl.BlockSpec((_BLK, _D), lambda i: (i, 0)),
            pl.BlockSpec((_D, 256), lambda i: (0, 0)),
            pl.BlockSpec((256,), lambda i: (0,)),
            pl.BlockSpec((256, 256), lambda i: (0, 0)),
            pl.BlockSpec((256,), lambda i: (0,)),
            pl.BlockSpec((256, _L), lambda i: (0, 0)),
            pl.BlockSpec((_L,), lambda i: (0,)),
            pl.BlockSpec((_K, _L), lambda i: (0, 0)),
        ],
        out_specs=[
            pl.BlockSpec((_BLK, _L), lambda i: (i, 0)),
            pl.BlockSpec((1, 1, _BLK), lambda i: (i, 0, 0)),
        ],
        out_shape=[
            jax.ShapeDtypeStruct((_B, _L), jnp.float32),
            jax.ShapeDtypeStruct((_NBLK, 1, _BLK), jnp.int32),
        ],
        scratch_shapes=[pltpu.VMEM((1, _K), jnp.float32)],
    )(x, w1, b1, w2, b2, w3, b3, cb)


def _sc_gather(codebook, idx3):
    mesh = plsc.VectorSubcoreMesh(core_axis_name="c", subcore_axis_name="s")

    @functools.partial(
        pl.kernel,
        out_type=jax.ShapeDtypeStruct((_B, _L), jnp.float32),
        mesh=mesh,
        compiler_params=pltpu.CompilerParams(use_tc_tiling_on_sc=False),
        scratch_types=[
            [pltpu.VMEM((_CH,), jnp.int32) for _ in range(_NCH)],
            pltpu.VMEM((_BPW, _L), jnp.float32),
            pltpu.SemaphoreType.DMA,
        ],
    )
    def k(cb_hbm, idx_hbm, out_hbm, idx_vs, rows_v, sem):
        wid = lax.axis_index("s") * 2 + lax.axis_index("c")
        base = wid * _BPW
        for j in range(_NCH):
            pltpu.sync_copy(idx_hbm.at[pl.ds(base + j * _CH, _CH)], idx_vs[j])
        copies = [
            pltpu.async_copy(
                cb_hbm.at[idx_vs[j]],
                rows_v.at[pl.ds(j * _CH, _CH)],
                sem,
            )
            for j in range(_NCH)
        ]
        for c in copies:
            c.wait()
        pltpu.sync_copy(rows_v, out_hbm.at[pl.ds(base, _BPW)])

    return k(codebook, idx3)


def _decode_body(ze_ref, zq_ref, w1_ref, b1_ref, w2_ref, b2_ref, w3_ref,
                 b3_ref, xrec_ref, loss_ref, acc_ref):
    i = pl.program_id(0)
    ze = ze_ref[...]
    zq = zq_ref[...]
    zst = ze + (zq - ze)
    h = jnp.maximum(jnp.dot(zst, w1_ref[...]) + b1_ref[...], 0.0)
    h = jnp.maximum(jnp.dot(h, w2_ref[...]) + b2_ref[...], 0.0)
    xrec_ref[...] = jnp.dot(h, w3_ref[...]) + b3_ref[...]

    diff = ze - zq
    part = jnp.sum(diff * diff)

    @pl.when(i == 0)
    def _():
        acc_ref[0] = 0.0

    acc_ref[0] += part

    @pl.when(i == _NBLK - 1)
    def _():
        loss_ref[...] = (acc_ref[0] / jnp.float32(_B * _L)).reshape(1, 1)


def _decode(ze, zq, w1, b1, w2, b2, w3, b3):
    return pl.pallas_call(
        _decode_body,
        grid=(_NBLK,),
        in_specs=[
            pl.BlockSpec((_BLK, _L), lambda i: (i, 0)),
            pl.BlockSpec((_BLK, _L), lambda i: (i, 0)),
            pl.BlockSpec((_L, 256), lambda i: (0, 0)),
            pl.BlockSpec((256,), lambda i: (0,)),
            pl.BlockSpec((256, 256), lambda i: (0, 0)),
            pl.BlockSpec((256,), lambda i: (0,)),
            pl.BlockSpec((256, _D), lambda i: (0, 0)),
            pl.BlockSpec((_D,), lambda i: (0,)),
        ],
        out_specs=[
            pl.BlockSpec((_BLK, _D), lambda i: (i, 0)),
            pl.BlockSpec((1, 1), lambda i: (0, 0)),
        ],
        out_shape=[
            jax.ShapeDtypeStruct((_B, _D), jnp.float32),
            jax.ShapeDtypeStruct((1, 1), jnp.float32),
        ],
        scratch_shapes=[pltpu.SMEM((1,), jnp.float32)],
    )(ze, zq, w1, b1, w2, b2, w3, b3)


def kernel(x, enc_w1, enc_b1, enc_w2, enc_b2, enc_w3, enc_b3,
           dec_w1, dec_b1, dec_w2, dec_b2, dec_w3, dec_b3, codebook):
    ze, idx3 = _encode_vq(x, enc_w1, enc_b1, enc_w2, enc_b2, enc_w3, enc_b3,
                          codebook)
    idx_flat = idx3.reshape(_B)
    zq = _sc_gather(codebook, idx_flat)
    xrec, loss = _decode(ze, zq, dec_w1, dec_b1, dec_w2, dec_b2,
                         dec_w3, dec_b3)
    return xrec, zq, jnp.reshape(loss, ()), idx_flat


# speedup vs baseline: 1.7414x; 1.7414x over previous
import functools

import jax
import jax.numpy as jnp
from jax import lax
from jax.experimental import pallas as pl
from jax.experimental.pallas import tpu as pltpu
from jax.experimental.pallas import tpu_sc as plsc

_B, _D, _L, _K = 9216, 768, 256, 8192
_BLK = 256
_NBLK = _B // _BLK

_NW = 32
_BPW = _B // _NW
_CH = 96
_NCH = _BPW // _CH


def _encode_vq_body(x_ref, w1_ref, b1_ref, w2_ref, b2_ref, w3_ref, b3_ref,
                    cb_ref, ze_ref, idx_ref, csum_ref):
    i = pl.program_id(0)

    @pl.when(i == 0)
    def _():
        cb = cb_ref[...]
        csum_ref[...] = jnp.sum(cb * cb, axis=1)[None, :]

    x = x_ref[...]
    h = jnp.maximum(jnp.dot(x, w1_ref[...]) + b1_ref[...], 0.0)
    h = jnp.maximum(jnp.dot(h, w2_ref[...]) + b2_ref[...], 0.0)
    z = jnp.dot(h, w3_ref[...]) + b3_ref[...]
    ze_ref[...] = z

    ab = lax.dot_general(z, cb_ref[...], (((1,), (1,)), ((), ())))
    rowsum = jnp.sum(z * z, axis=1, keepdims=True)
    d2 = (rowsum - 2.0 * ab) + csum_ref[...]
    m = jnp.min(d2, axis=1, keepdims=True)
    col = lax.broadcasted_iota(jnp.int32, (_BLK, _K), 1)
    idx = jnp.min(jnp.where(d2 == m, col, _K), axis=1)
    idx_ref[...] = idx[None, None, :]


def _encode_vq(x, w1, b1, w2, b2, w3, b3, cb):
    return pl.pallas_call(
        _encode_vq_body,
        grid=(_NBLK,),
        in_specs=[
            pl.BlockSpec((_BLK, _D), lambda i: (i, 0)),
            pl.BlockSpec((_D, 256), lambda i: (0, 0)),
            pl.BlockSpec((256,), lambda i: (0,)),
            pl.BlockSpec((256, 256), lambda i: (0, 0)),
            pl.BlockSpec((256,), lambda i: (0,)),
            pl.BlockSpec((256, _L), lambda i: (0, 0)),
            pl.BlockSpec((_L,), lambda i: (0,)),
            pl.BlockSpec((_K, _L), lambda i: (0, 0)),
        ],
        out_specs=[
            pl.BlockSpec((_BLK, _L), lambda i: (i, 0)),
            pl.BlockSpec((1, 1, _BLK), lambda i: (i, 0, 0)),
        ],
        out_shape=[
            jax.ShapeDtypeStruct((_B, _L), jnp.float32),
            jax.ShapeDtypeStruct((_NBLK, 1, _BLK), jnp.int32),
        ],
        scratch_shapes=[pltpu.VMEM((1, _K), jnp.float32)],
    )(x, w1, b1, w2, b2, w3, b3, cb)


_KH = _K // 2
_SB = 96
_NSB = _BPW // _SB


def _sc_gather(codebook, idx_flat):
    mesh = plsc.VectorSubcoreMesh(core_axis_name="c", subcore_axis_name="s")

    @functools.partial(
        pl.kernel,
        out_type=jax.ShapeDtypeStruct((_B, _L), jnp.float32),
        mesh=mesh,
        compiler_params=pltpu.CompilerParams(use_tc_tiling_on_sc=False,
                                             needs_layout_passes=False),
        scratch_types=[
            pltpu.VMEM_SHARED((_KH, _L), jnp.float32),
            pltpu.VMEM_SHARED((_B,), jnp.int32),
            pltpu.VMEM((_SB, _L), jnp.float32),
            pltpu.VMEM((_SB, _L), jnp.float32),
            pltpu.VMEM((_SB,), jnp.int32),
            pltpu.SMEM((_BPW,), jnp.int32),
            pltpu.SemaphoreType.DMA,
        ],
    )
    def k(cb_hbm, idx_hbm, out_hbm, table_sp, idx_sp, buf_a, buf_b, idx_v,
          idx_sm, sem):
        sid = lax.axis_index("s")
        wid = sid * 2 + lax.axis_index("c")
        base = wid * _BPW

        @pl.when(sid == 0)
        def _():
            pltpu.sync_copy(idx_hbm, idx_sp)

        plsc.subcore_barrier()
        pltpu.sync_copy(idx_sp.at[pl.ds(base, _BPW)], idx_sm)

        for s in range(_NSB):
            @pl.when(sid == 0)
            def _():
                pltpu.sync_copy(cb_hbm.at[pl.ds(0, _KH)], table_sp)

            plsc.subcore_barrier()
            pltpu.sync_copy(idx_hbm.at[pl.ds(base + s * _SB, _SB)], idx_v)
            for t in range(_SB // 16):
                sl = pl.ds(t * 16, 16)
                idx_v[sl] = jnp.minimum(idx_v[sl], _KH - 1)
            pltpu.async_copy(table_sp.at[idx_v], buf_a, sem).wait()
            plsc.subcore_barrier()

            @pl.when(sid == 0)
            def _():
                pltpu.sync_copy(cb_hbm.at[pl.ds(_KH, _KH)], table_sp)

            plsc.subcore_barrier()
            pltpu.sync_copy(idx_hbm.at[pl.ds(base + s * _SB, _SB)], idx_v)
            for t in range(_SB // 16):
                sl = pl.ds(t * 16, 16)
                idx_v[sl] = jnp.maximum(idx_v[sl] - _KH, 0)
            pltpu.async_copy(table_sp.at[idx_v], buf_b, sem).wait()

            def body(i, carry):
                @pl.when(idx_sm[s * _SB + i] >= _KH)
                def _():
                    for c in range(_L // 16):
                        cs = pl.ds(c * 16, 16)
                        buf_a[i, cs] = buf_b[i, cs]

                return carry

            lax.fori_loop(0, _SB, body, 0)

            pltpu.sync_copy(buf_a, out_hbm.at[pl.ds(base + s * _SB, _SB)])
            plsc.subcore_barrier()

    return k(codebook, idx_flat)


def _decode_body(ze_ref, zq_ref, w1_ref, b1_ref, w2_ref, b2_ref, w3_ref,
                 b3_ref, xrec_ref, loss_ref, acc_ref):
    i = pl.program_id(0)
    ze = ze_ref[...]
    zq = zq_ref[...]
    zst = ze + (zq - ze)
    h = jnp.maximum(jnp.dot(zst, w1_ref[...]) + b1_ref[...], 0.0)
    h = jnp.maximum(jnp.dot(h, w2_ref[...]) + b2_ref[...], 0.0)
    xrec_ref[...] = jnp.dot(h, w3_ref[...]) + b3_ref[...]

    diff = ze - zq
    part = jnp.sum(diff * diff)

    @pl.when(i == 0)
    def _():
        acc_ref[0] = 0.0

    acc_ref[0] += part

    @pl.when(i == _NBLK - 1)
    def _():
        loss_ref[...] = (acc_ref[0] / jnp.float32(_B * _L)).reshape(1, 1)


def _decode(ze, zq, w1, b1, w2, b2, w3, b3):
    return pl.pallas_call(
        _decode_body,
        grid=(_NBLK,),
        in_specs=[
            pl.BlockSpec((_BLK, _L), lambda i: (i, 0)),
            pl.BlockSpec((_BLK, _L), lambda i: (i, 0)),
            pl.BlockSpec((_L, 256), lambda i: (0, 0)),
            pl.BlockSpec((256,), lambda i: (0,)),
            pl.BlockSpec((256, 256), lambda i: (0, 0)),
            pl.BlockSpec((256,), lambda i: (0,)),
            pl.BlockSpec((256, _D), lambda i: (0, 0)),
            pl.BlockSpec((_D,), lambda i: (0,)),
        ],
        out_specs=[
            pl.BlockSpec((_BLK, _D), lambda i: (i, 0)),
            pl.BlockSpec((1, 1), lambda i: (0, 0)),
        ],
        out_shape=[
            jax.ShapeDtypeStruct((_B, _D), jnp.float32),
            jax.ShapeDtypeStruct((1, 1), jnp.float32),
        ],
        scratch_shapes=[pltpu.SMEM((1,), jnp.float32)],
    )(ze, zq, w1, b1, w2, b2, w3, b3)


def kernel(x, enc_w1, enc_b1, enc_w2, enc_b2, enc_w3, enc_b3,
           dec_w1, dec_b1, dec_w2, dec_b2, dec_w3, dec_b3, codebook):
    ze, idx3 = _encode_vq(x, enc_w1, enc_b1, enc_w2, enc_b2, enc_w3, enc_b3,
                          codebook)
    idx_flat = idx3.reshape(_B)
    zq = _sc_gather(codebook, idx_flat)
    xrec, loss = _decode(ze, zq, dec_w1, dec_b1, dec_w2, dec_b2,
                         dec_w3, dec_b3)
    return xrec, zq, jnp.reshape(loss, ()), idx_flat
